# baseline (speedup 1.0000x reference)
import functools

import jax
import jax.numpy as jnp
from jax import lax
from jax.experimental import pallas as pl
from jax.experimental.pallas import tpu as pltpu
from jax.experimental.pallas import tpu_sc as plsc

VOCAB = 1000000
EMBED_DIM = 64
BATCH = 4096
SEQ = 200

NC = 2
NS = 16
NW = NC * NS
RPW = BATCH // NW

CHUNK_A = 128
CHUNK_B = SEQ - CHUNK_A


def _encoder_kernel(ids_hbm, table_hbm, out_hbm,
                    idx_a, idx_b, rows_a, rows_b, out_v, sem):
    wid = lax.axis_index("s") * NC + lax.axis_index("c")
    base = wid * RPW

    inv = jnp.float32(1.0 / SEQ)

    def row_body(r, carry):
        off = (base + r) * SEQ
        pltpu.sync_copy(ids_hbm.at[pl.ds(off, CHUNK_A)], idx_a)
        pltpu.sync_copy(ids_hbm.at[pl.ds(off + CHUNK_A, CHUNK_B)], idx_b)
        cp_a = pltpu.async_copy(table_hbm.at[idx_a], rows_a, sem)
        cp_b = pltpu.async_copy(table_hbm.at[idx_b], rows_b, sem)
        cp_a.wait()
        cp_b.wait()

        def acc_a(j, accs):
            return tuple(
                accs[k] + rows_a[j, pl.ds(16 * k, 16)] for k in range(4))

        def acc_b(j, accs):
            return tuple(
                accs[k] + rows_b[j, pl.ds(16 * k, 16)] for k in range(4))

        zeros = tuple(jnp.zeros((16,), jnp.float32) for _ in range(4))
        accs = lax.fori_loop(0, CHUNK_A, acc_a, zeros, unroll=8)
        accs = lax.fori_loop(0, CHUNK_B, acc_b, accs, unroll=8)
        for k in range(4):
            out_v[r, pl.ds(16 * k, 16)] = accs[k] * inv
        return carry

    lax.fori_loop(0, RPW, row_body, 0)
    pltpu.sync_copy(out_v, out_hbm.at[pl.ds(base, RPW)])


@functools.partial(jax.jit, static_argnames=())
def kernel(text_ids, table):
    ids_flat = text_ids.reshape(-1).astype(jnp.int32)
    mesh = plsc.VectorSubcoreMesh(core_axis_name="c", subcore_axis_name="s")
    k = functools.partial(
        pl.kernel,
        mesh=mesh,
        out_type=jax.ShapeDtypeStruct((BATCH, EMBED_DIM), jnp.float32),
        scratch_types=[
            pltpu.VMEM((CHUNK_A,), jnp.int32),
            pltpu.VMEM((CHUNK_B,), jnp.int32),
            pltpu.VMEM((CHUNK_A, EMBED_DIM), jnp.float32),
            pltpu.VMEM((CHUNK_B, EMBED_DIM), jnp.float32),
            pltpu.VMEM((RPW, EMBED_DIM), jnp.float32),
            pltpu.SemaphoreType.DMA,
        ],
        compiler_params=pltpu.CompilerParams(use_tc_tiling_on_sc=False),
    )(_encoder_kernel)
    return k(ids_flat, table)


# speedup vs baseline: 1.2855x; 1.2855x over previous
import functools

import jax
import jax.numpy as jnp
from jax import lax
from jax.experimental import pallas as pl
from jax.experimental.pallas import tpu as pltpu
from jax.experimental.pallas import tpu_sc as plsc

VOCAB = 1000000
EMBED_DIM = 64
BATCH = 4096
SEQ = 200

NC = 2
NS = 16
NW = NC * NS
RPW = BATCH // NW

CHUNK_A = 128
CHUNK_B = SEQ - CHUNK_A


def _encoder_kernel(ids_hbm, table_hbm, out_hbm,
                    idx_all, rows0, rows1, out_v, sem0, sem1):
    wid = lax.axis_index("s") * NC + lax.axis_index("c")
    base = wid * RPW

    inv = jnp.float32(1.0 / SEQ)
    rows = (rows0, rows1)
    sems = (sem0, sem1)

    pltpu.sync_copy(ids_hbm.at[pl.ds(base * SEQ, RPW * SEQ)], idx_all)

    def fire(r, slot):
        off = r * SEQ
        pltpu.async_copy(
            table_hbm.at[idx_all.at[pl.ds(off, CHUNK_A)]],
            rows[slot].at[pl.ds(0, CHUNK_A)], sems[slot])
        pltpu.async_copy(
            table_hbm.at[idx_all.at[pl.ds(off + CHUNK_A, CHUNK_B)]],
            rows[slot].at[pl.ds(CHUNK_A, CHUNK_B)], sems[slot])

    def wait(slot):
        pltpu.make_async_copy(
            table_hbm.at[idx_all.at[pl.ds(0, CHUNK_A)]],
            rows[slot].at[pl.ds(0, CHUNK_A)], sems[slot]).wait()
        pltpu.make_async_copy(
            table_hbm.at[idx_all.at[pl.ds(0, CHUNK_B)]],
            rows[slot].at[pl.ds(CHUNK_A, CHUNK_B)], sems[slot]).wait()

    def accum(r, slot):
        buf = rows[slot]

        def acc_body(j, accs):
            return tuple(
                accs[k] + buf[j, pl.ds(16 * k, 16)] for k in range(4))

        zeros = tuple(jnp.zeros((16,), jnp.float32) for _ in range(4))
        accs = lax.fori_loop(0, SEQ, acc_body, zeros, unroll=8)
        for k in range(4):
            out_v[r, pl.ds(16 * k, 16)] = accs[k] * inv

    fire(0, 0)

    def outer(rr, carry):
        r0 = 2 * rr
        fire(r0 + 1, 1)
        wait(0)
        accum(r0, 0)

        @pl.when(r0 + 2 < RPW)
        def _():
            fire(r0 + 2, 0)

        wait(1)
        accum(r0 + 1, 1)
        return carry

    lax.fori_loop(0, RPW // 2, outer, 0)
    pltpu.sync_copy(out_v, out_hbm.at[pl.ds(base, RPW)])


def kernel(text_ids, table):
    ids_flat = text_ids.reshape(-1).astype(jnp.int32)
    mesh = plsc.VectorSubcoreMesh(core_axis_name="c", subcore_axis_name="s")
    k = functools.partial(
        pl.kernel,
        mesh=mesh,
        out_type=jax.ShapeDtypeStruct((BATCH, EMBED_DIM), jnp.float32),
        scratch_types=[
            pltpu.VMEM((RPW * SEQ,), jnp.int32),
            pltpu.VMEM((SEQ, EMBED_DIM), jnp.float32),
            pltpu.VMEM((SEQ, EMBED_DIM), jnp.float32),
            pltpu.VMEM((RPW, EMBED_DIM), jnp.float32),
            pltpu.SemaphoreType.DMA,
            pltpu.SemaphoreType.DMA,
        ],
        compiler_params=pltpu.CompilerParams(use_tc_tiling_on_sc=False),
    )(_encoder_kernel)
    return k(ids_flat, table)
